# Initial kernel scaffold; baseline (speedup 1.0000x reference)
import jax
import jax.numpy as jnp
from jax.experimental import pallas as pl


def kernel(state, edge_index, Wg, bg, W1, b1, W2, b2, Wm, bm, Ws, bs):
    raise NotImplementedError("write your pallas kernel here")


# speedup vs baseline: 19.2022x; 19.2022x over previous
import functools

import numpy as np
import jax
import jax.numpy as jnp
from jax import lax
from jax.experimental import pallas as pl
from jax.experimental.pallas import tpu as pltpu, tpu_sc as plsc

_N = 14336
_E = 458752
_IN = 128
_HID = 32
_AD = 14
_B = _N // _AD
_NP = _AD * _AD
_NE = _AD * (_AD - 1)

_NSC = 2
_NT = 16
_E32 = _E // (_NSC * _NT)
_HF = _IN // 2
_EC = _E // _NT
_CH = 128
_NCH = _EC // _CH
_RPT = _N // _NT

_ROWS_A = 56
_RB = _N // _ROWS_A
_G = 64
_GSTEPS = _B // _G

_MASK = np.ones((1, _NP), np.float32)
_MASK[0, :: _AD + 1] = 0.0


def _eps_consts():
    eps182 = jax.random.normal(jax.random.key(42), (_B, _NE), dtype=jnp.float32)
    eps_r = eps182.reshape(_B, _AD - 1, _AD)
    buf = jnp.concatenate(
        [eps_r, jnp.zeros((_B, _AD - 1, 1), jnp.float32)], axis=2)
    eps_grid = jnp.concatenate(
        [jnp.zeros((_B, 1), jnp.float32), buf.reshape(_B, _NP - 1)], axis=1)
    lpc = ((-0.5 * eps182 * eps182).sum(axis=1)
           - _NE * 0.5 * np.log(2.0 * np.pi)).reshape(_B, 1)
    return eps_grid, lpc


def _deg_body(edst_hbm, out_hbm, idx_v, deg_v):
    c = lax.axis_index("c")
    s = lax.axis_index("s")
    wid = s * _NSC + c
    pltpu.sync_copy(edst_hbm.at[wid], idx_v)
    z16 = jnp.zeros((16,), jnp.float32)

    @pl.loop(0, _N // 16)
    def _zero(i):
        deg_v[pl.ds(i * 16, 16)] = z16

    ones16 = jnp.ones((16,), jnp.float32)

    @pl.loop(0, _E32 // 16)
    def _acc(i):
        idx = idx_v[pl.ds(i * 16, 16)]
        plsc.addupdate_scatter(deg_v, [idx], ones16)

    pltpu.sync_copy(deg_v, out_hbm.at[wid])


@functools.cache
def _deg_call():
    return pl.kernel(
        _deg_body,
        out_type=jax.ShapeDtypeStruct((_NSC * _NT, _N), jnp.float32),
        mesh=plsc.VectorSubcoreMesh(core_axis_name="c", subcore_axis_name="s",
                                    num_cores=_NSC, num_subcores=_NT),
        scratch_types=[
            pltpu.VMEM((_E32,), jnp.int32),
            pltpu.VMEM((_N,), jnp.float32),
        ],
        compiler_params=pltpu.CompilerParams(needs_layout_passes=False,
                                             use_tc_tiling_on_sc=False),
    )


def _scat_body(y0_hbm, y1_hbm, esrc_hbm, edst_hbm, out_hbm, src_v, dst_v,
               rows_v, z_sh, sem):
    c = lax.axis_index("c")
    s = lax.axis_index("s")
    pltpu.sync_copy(esrc_hbm.at[s], src_v)
    pltpu.sync_copy(edst_hbm.at[s], dst_v)

    z16 = jnp.zeros((16,), jnp.float32)

    @pl.loop(0, _CH)
    def _zrow(i):
        @pl.loop(0, _HF // 16)
        def _zcol(j):
            rows_v[i, pl.ds(j * 16, 16)] = z16

    @pl.loop(0, _RPT // _CH)
    def _zacc(k):
        pltpu.sync_copy(rows_v, z_sh.at[pl.ds(s * _RPT + k * _CH, _CH)])

    plsc.subcore_barrier()

    @pl.loop(0, _NCH)
    def _edges(j):
        @pl.when(c == 0)
        def _g0():
            pltpu.async_copy(y0_hbm.at[src_v.at[j]], rows_v, sem).wait()

        @pl.when(c == 1)
        def _g1():
            pltpu.async_copy(y1_hbm.at[src_v.at[j]], rows_v, sem).wait()

        pltpu.sync_copy(rows_v, z_sh.at[dst_v.at[j]], add=True)

    plsc.subcore_barrier()

    @pl.loop(0, _RPT // _CH)
    def _dump(k):
        r = s * _RPT + k * _CH
        pltpu.sync_copy(z_sh.at[pl.ds(r, _CH)], out_hbm.at[c, pl.ds(r, _CH)])


@functools.cache
def _scat_call():
    return pl.kernel(
        _scat_body,
        out_type=jax.ShapeDtypeStruct((_NSC, _N, _HF), jnp.float32),
        mesh=plsc.VectorSubcoreMesh(core_axis_name="c", subcore_axis_name="s",
                                    num_cores=_NSC, num_subcores=_NT),
        scratch_types=[
            pltpu.VMEM((_NCH, _CH), jnp.int32),
            pltpu.VMEM((_NCH, _CH), jnp.int32),
            pltpu.VMEM((_CH, _HF), jnp.float32),
            pltpu.VMEM_SHARED((_N, _HF), jnp.float32),
            pltpu.SemaphoreType.DMA,
        ],
        compiler_params=pltpu.CompilerParams(needs_layout_passes=False,
                                             use_tc_tiling_on_sc=False),
    )


def _a_body(degp_ref, state_ref, wg_ref, y0_ref, y1_ref, dis_ref):
    deg = jnp.sum(degp_ref[...], axis=0) + 1.0
    dis = lax.rsqrt(deg)
    xw = jnp.dot(state_ref[...], wg_ref[...], preferred_element_type=jnp.float32)
    y = xw * dis[:, None]
    y0_ref[...] = y[:, :_HF]
    y1_ref[...] = y[:, _HF:]
    dis_ref[...] = dis[:, None]


_a_call = pl.pallas_call(
    _a_body,
    grid=(_ROWS_A,),
    in_specs=[
        pl.BlockSpec((_NSC * _NT, _RB), lambda i: (0, i)),
        pl.BlockSpec((_RB, _IN), lambda i: (i, 0)),
        pl.BlockSpec((_IN, _IN), lambda i: (0, 0)),
    ],
    out_specs=[
        pl.BlockSpec((_RB, _HF), lambda i: (i, 0)),
        pl.BlockSpec((_RB, _HF), lambda i: (i, 0)),
        pl.BlockSpec((_RB, 1), lambda i: (i, 0)),
    ],
    out_shape=[
        jax.ShapeDtypeStruct((_N, _HF), jnp.float32),
        jax.ShapeDtypeStruct((_N, _HF), jnp.float32),
        jax.ShapeDtypeStruct((_N, 1), jnp.float32),
    ],
)


def _b_body(z_ref, y0_ref, y1_ref, dis_ref, state_ref, bg_ref, w1_ref, b1_ref,
            pq_ref):
    zsum = jnp.concatenate(
        [z_ref[0] + y0_ref[...], z_ref[1] + y1_ref[...]], axis=-1)
    out = zsum * dis_ref[...] + bg_ref[...]
    x = jnp.maximum(out, 0.0) + state_ref[...]
    pq_ref[...] = (jnp.dot(x, w1_ref[...], preferred_element_type=jnp.float32)
                   + b1_ref[...])


_b_call = pl.pallas_call(
    _b_body,
    grid=(_ROWS_A,),
    in_specs=[
        pl.BlockSpec((_NSC, _RB, _HF), lambda i: (0, i, 0)),
        pl.BlockSpec((_RB, _HF), lambda i: (i, 0)),
        pl.BlockSpec((_RB, _HF), lambda i: (i, 0)),
        pl.BlockSpec((_RB, 1), lambda i: (i, 0)),
        pl.BlockSpec((_RB, _IN), lambda i: (i, 0)),
        pl.BlockSpec((1, _IN), lambda i: (0, 0)),
        pl.BlockSpec((_IN, 2 * _HID), lambda i: (0, 0)),
        pl.BlockSpec((1, 2 * _HID), lambda i: (0, 0)),
    ],
    out_specs=pl.BlockSpec((_RB, 2 * _HID), lambda i: (i, 0)),
    out_shape=jax.ShapeDtypeStruct((_N, 2 * _HID), jnp.float32),
)


def _softplus(x):
    return jnp.maximum(x, 0.0) + jnp.log1p(jnp.exp(-jnp.abs(x)))


def _c_body(pq_ref, w2_ref, b2_ref, wm_ref, bm_ref, ws_ref, bs_ref,
            eps_ref, cc_ref, mask_ref, act_ref, lp_ref):
    pq = pq_ref[...].reshape(_G, _AD, 2 * _HID)
    p = pq[:, :, :_HID]
    q = pq[:, :, _HID:]
    sgrid = p[:, :, None, :] + q[:, None, :, :]
    h = sgrid.reshape(_G * _NP, _HID)
    h = jnp.where(h > 0, h, 0.01 * h)
    h = jnp.dot(h, w2_ref[...], preferred_element_type=jnp.float32) + b2_ref[...]
    h = jnp.where(h > 0, h, 0.01 * h)
    m = jnp.sum(h * wm_ref[...], axis=1) + bm_ref[0, 0]
    sv = jnp.sum(h * ws_ref[...], axis=1) + bs_ref[0, 0]
    mu = _softplus(m).reshape(_G, _NP)
    sd = _softplus(sv).reshape(_G, _NP)
    act_ref[...] = mu + sd * eps_ref[...]
    lp_ref[...] = cc_ref[...] - jnp.sum(jnp.log(sd) * mask_ref[...], axis=1,
                                        keepdims=True)


_c_call = pl.pallas_call(
    _c_body,
    grid=(_GSTEPS,),
    in_specs=[
        pl.BlockSpec((_G * _AD, 2 * _HID), lambda i: (i, 0)),
        pl.BlockSpec((_HID, _HID), lambda i: (0, 0)),
        pl.BlockSpec((1, _HID), lambda i: (0, 0)),
        pl.BlockSpec((1, _HID), lambda i: (0, 0)),
        pl.BlockSpec((1, 1), lambda i: (0, 0)),
        pl.BlockSpec((1, _HID), lambda i: (0, 0)),
        pl.BlockSpec((1, 1), lambda i: (0, 0)),
        pl.BlockSpec((_G, _NP), lambda i: (i, 0)),
        pl.BlockSpec((_G, 1), lambda i: (i, 0)),
        pl.BlockSpec((1, _NP), lambda i: (0, 0)),
    ],
    out_specs=[
        pl.BlockSpec((_G, _NP), lambda i: (i, 0)),
        pl.BlockSpec((_G, 1), lambda i: (i, 0)),
    ],
    out_shape=[
        jax.ShapeDtypeStruct((_B, _NP), jnp.float32),
        jax.ShapeDtypeStruct((_B, 1), jnp.float32),
    ],
)


def kernel(state, edge_index, Wg, bg, W1, b1, W2, b2, Wm, bm, Ws, bs):
    edst32 = edge_index[1].reshape(_NSC * _NT, _E32)
    esrc3 = edge_index[0].reshape(_NT, _NCH, _CH)
    edst3 = edge_index[1].reshape(_NT, _NCH, _CH)

    degp = _deg_call()(edst32)
    y0, y1, dis = _a_call(degp, state, Wg)
    z2 = _scat_call()(y0, y1, esrc3, edst3)

    w1cat = jnp.concatenate([W1[:_IN], W1[_IN:]], axis=1)
    b1cat = jnp.concatenate([jnp.zeros((_HID,), jnp.float32), b1]).reshape(1, -1)
    pq = _b_call(z2, y0, y1, dis, state, bg.reshape(1, _IN), w1cat, b1cat)

    eps_grid, lpc = _eps_consts()
    act_grid, lp = _c_call(
        pq, W2, b2.reshape(1, _HID), Wm.reshape(1, _HID),
        bm.reshape(1, 1), Ws.reshape(1, _HID), bs.reshape(1, 1),
        eps_grid, lpc, jnp.asarray(_MASK))

    action = act_grid[:, 1:].reshape(_B, _AD - 1, _AD + 1)[:, :, :_AD]
    action = action.reshape(_B, _NE)
    return (action, jnp.squeeze(lp, -1))
